# baseline (speedup 1.0000x reference)
import functools

import jax
import jax.numpy as jnp
from jax import lax
from jax.experimental import pallas as pl
from jax.experimental.pallas import tpu as pltpu
from jax.experimental.pallas import tpu_sc as plsc

B = 16384
N_CAT = 26
N_CONT = 16
CARD = 100000
DIM = 32
N_TOK = N_CAT + N_CONT

NC = 2
NS = 16
NW = NC * NS
ROWS_W = B // NW
CB = 32
NCHUNK = ROWS_W // CB
IDX_N = CB * N_CAT
GSUB = 8
GROWS = IDX_N // GSUB
LANES = 16


def _sc_body(xcat_hbm, xcont_hbm, table_hbm, w_hbm, bias_hbm, offs_hbm,
             out_hbm, idx_v, cat_v, cont_v, xc_v, offs_v, w_v, bias_v,
             gsem):
  c = lax.axis_index("c")
  s = lax.axis_index("s")
  wid = s * NC + c
  base_row = wid * ROWS_W

  pltpu.sync_copy(offs_hbm, offs_v)
  pltpu.sync_copy(w_hbm, w_v)
  pltpu.sync_copy(bias_hbm, bias_v)

  def chunk_body(g, carry):
    b0 = base_row + g * CB
    pltpu.sync_copy(xcat_hbm.at[pl.ds(b0 * N_CAT, IDX_N)], idx_v)
    for k in range(IDX_N // LANES):
      sl = pl.ds(k * LANES, LANES)
      idx_v[sl] = idx_v[sl] + offs_v[sl]
    copies = []
    for j in range(GSUB):
      cp = pltpu.make_async_copy(
          table_hbm.at[idx_v.at[pl.ds(j * GROWS, GROWS)]],
          cat_v.at[pl.ds(j * GROWS, GROWS)],
          gsem)
      cp.start()
      copies.append(cp)
    pltpu.sync_copy(xcont_hbm.at[pl.ds(b0, CB)], xc_v)
    for bl in range(CB):
      vrow = xc_v[bl, :]
      for j in range(N_CONT):
        vx = lax.gather(
            vrow, jnp.full((LANES, 1), j, jnp.int32),
            dimension_numbers=lax.GatherDimensionNumbers(
                offset_dims=(), collapsed_slice_dims=(0,),
                start_index_map=(0,)),
            slice_sizes=(1,),
            mode=lax.GatherScatterMode.PROMISE_IN_BOUNDS)
        for d in range(DIM // LANES):
          sl = pl.ds(d * LANES, LANES)
          cont_v[bl, j, sl] = vx * w_v[j, sl] + bias_v[j, sl]
    for cp in copies:
      cp.wait()
    for i in range(CB):
      pltpu.sync_copy(cat_v.at[pl.ds(i * N_CAT, N_CAT)],
                      out_hbm.at[b0 + i, pl.ds(0, N_CAT)])
    pltpu.sync_copy(cont_v,
                    out_hbm.at[pl.ds(b0, CB), pl.ds(N_CAT, N_CONT)])
    return carry

  lax.fori_loop(0, NCHUNK, chunk_body, 0)


_sc_kernel = functools.partial(
    pl.kernel,
    mesh=plsc.VectorSubcoreMesh(core_axis_name="c", subcore_axis_name="s"),
    compiler_params=pltpu.CompilerParams(use_tc_tiling_on_sc=False),
    out_type=jax.ShapeDtypeStruct((B, N_TOK, DIM), jnp.float32),
    scratch_types=[
        pltpu.VMEM((IDX_N,), jnp.int32),
        pltpu.VMEM((IDX_N, DIM), jnp.float32),
        pltpu.VMEM((CB, N_CONT, DIM), jnp.float32),
        pltpu.VMEM((CB, N_CONT), jnp.float32),
        pltpu.VMEM((IDX_N,), jnp.int32),
        pltpu.VMEM((N_CONT, DIM), jnp.float32),
        pltpu.VMEM((N_CONT, DIM), jnp.float32),
        pltpu.SemaphoreType.DMA,
    ],
)(_sc_body)


@jax.jit
def kernel(x_cat, x_cont, cat_table, cont_W, cont_b):
  xcat_flat = x_cat.astype(jnp.int32).reshape(B * N_CAT)
  offs_tile = jnp.tile(jnp.arange(N_CAT, dtype=jnp.int32) * CARD, CB)
  return _sc_kernel(xcat_flat, x_cont, cat_table, cont_W, cont_b, offs_tile)


# speedup vs baseline: 1.0259x; 1.0259x over previous
import functools

import jax
import jax.numpy as jnp
from jax import lax
from jax.experimental import pallas as pl
from jax.experimental.pallas import tpu as pltpu
from jax.experimental.pallas import tpu_sc as plsc

B = 16384
N_CAT = 26
N_CONT = 16
CARD = 100000
DIM = 32
N_TOK = N_CAT + N_CONT

NC = 2
NS = 16
NW = NC * NS
ROWS_W = B // NW
CB = 32
NCHUNK = ROWS_W // CB
IDX_N = CB * N_CAT
GSUB = 8
GROWS = IDX_N // GSUB
LANES = 16


def _sc_body(xcat_hbm, xcont_hbm, table_hbm, w_hbm, bias_hbm, offs_hbm,
             out_hbm, idx_v, cat_v, cont_v, xc_v, offs_v, w_v, bias_v,
             gsem, lsem0, lsem1, osem0, osem1):
  c = lax.axis_index("c")
  s = lax.axis_index("s")
  wid = s * NC + c
  base_row = wid * ROWS_W
  lsems = (lsem0, lsem1)
  osems = (osem0, osem1)

  pltpu.sync_copy(offs_hbm, offs_v)
  pltpu.sync_copy(w_hbm, w_v)
  pltpu.sync_copy(bias_hbm, bias_v)

  def load_copies(b0, p, sem):
    return (
        pltpu.make_async_copy(
            xcat_hbm.at[pl.ds(b0 * N_CAT, IDX_N)], idx_v.at[p], sem),
        pltpu.make_async_copy(
            xcont_hbm.at[pl.ds(b0, CB)], xc_v.at[p], sem),
    )

  def out_copies(b0, p, sem):
    cps = [
        pltpu.make_async_copy(
            cat_v.at[p, pl.ds(i * N_CAT, N_CAT)],
            out_hbm.at[b0 + i, pl.ds(0, N_CAT)], sem)
        for i in range(CB)
    ]
    cps.append(pltpu.make_async_copy(
        cont_v.at[p],
        out_hbm.at[pl.ds(b0, CB), pl.ds(N_CAT, N_CONT)], sem))
    return cps

  for cp in load_copies(base_row, 0, lsem0):
    cp.start()

  def pair_body(go, carry):
    for p in (0, 1):
      g = go * 2 + p
      b0 = base_row + g * CB
      for cp in load_copies(b0, p, lsems[p]):
        cp.wait()
      @pl.when(g >= 2)
      def _():
        for cp in out_copies(b0, p, osems[p]):
          cp.wait()
      @pl.when(g + 1 < NCHUNK)
      def _():
        for cp in load_copies(b0 + CB, 1 - p, lsems[1 - p]):
          cp.start()
      for k in range(IDX_N // LANES):
        sl = pl.ds(k * LANES, LANES)
        idx_v[p, sl] = idx_v[p, sl] + offs_v[sl]
      gcps = []
      for j in range(GSUB):
        cp = pltpu.make_async_copy(
            table_hbm.at[idx_v.at[p, pl.ds(j * GROWS, GROWS)]],
            cat_v.at[p, pl.ds(j * GROWS, GROWS)],
            gsem)
        cp.start()
        gcps.append(cp)
      for bl in range(CB):
        vrow = xc_v[p, bl, :]
        for j in range(N_CONT):
          vx = lax.gather(
              vrow, jnp.full((LANES, 1), j, jnp.int32),
              dimension_numbers=lax.GatherDimensionNumbers(
                  offset_dims=(), collapsed_slice_dims=(0,),
                  start_index_map=(0,)),
              slice_sizes=(1,),
              mode=lax.GatherScatterMode.PROMISE_IN_BOUNDS)
          for d in range(DIM // LANES):
            sl = pl.ds(d * LANES, LANES)
            cont_v[p, bl, j, sl] = vx * w_v[j, sl] + bias_v[j, sl]
      for cp in gcps:
        cp.wait()
      for cp in out_copies(b0, p, osems[p]):
        cp.start()
    return carry

  lax.fori_loop(0, NCHUNK // 2, pair_body, 0)

  for p in (0, 1):
    b0 = base_row + (NCHUNK - 2 + p) * CB
    for cp in out_copies(b0, p, osems[p]):
      cp.wait()


_sc_kernel = functools.partial(
    pl.kernel,
    mesh=plsc.VectorSubcoreMesh(core_axis_name="c", subcore_axis_name="s"),
    compiler_params=pltpu.CompilerParams(use_tc_tiling_on_sc=False),
    out_type=jax.ShapeDtypeStruct((B, N_TOK, DIM), jnp.float32),
    scratch_types=[
        pltpu.VMEM((2, IDX_N), jnp.int32),
        pltpu.VMEM((2, IDX_N, DIM), jnp.float32),
        pltpu.VMEM((2, CB, N_CONT, DIM), jnp.float32),
        pltpu.VMEM((2, CB, N_CONT), jnp.float32),
        pltpu.VMEM((IDX_N,), jnp.int32),
        pltpu.VMEM((N_CONT, DIM), jnp.float32),
        pltpu.VMEM((N_CONT, DIM), jnp.float32),
        pltpu.SemaphoreType.DMA,
        pltpu.SemaphoreType.DMA,
        pltpu.SemaphoreType.DMA,
        pltpu.SemaphoreType.DMA,
        pltpu.SemaphoreType.DMA,
    ],
)(_sc_body)


@jax.jit
def kernel(x_cat, x_cont, cat_table, cont_W, cont_b):
  xcat_flat = x_cat.astype(jnp.int32).reshape(B * N_CAT)
  offs_tile = jnp.tile(jnp.arange(N_CAT, dtype=jnp.int32) * CARD, CB)
  return _sc_kernel(xcat_flat, x_cont, cat_table, cont_W, cont_b, offs_tile)


# speedup vs baseline: 1.0661x; 1.0392x over previous
import functools

import jax
import jax.numpy as jnp
from jax import lax
from jax.experimental import pallas as pl
from jax.experimental.pallas import tpu as pltpu
from jax.experimental.pallas import tpu_sc as plsc

B = 16384
N_CAT = 26
N_CONT = 16
CARD = 100000
DIM = 32
N_TOK = N_CAT + N_CONT

NC = 2
NS = 16
NW = NC * NS
BW = B // NW
GC = 128
NG = BW // GC
NST = N_CAT * NG
SUB = DIM // 8
LANES = 16


def _splat(vec, i):
  return lax.gather(
      vec, jnp.full((LANES, 1), i, jnp.int32),
      dimension_numbers=lax.GatherDimensionNumbers(
          offset_dims=(), collapsed_slice_dims=(0,), start_index_map=(0,)),
      slice_sizes=(1,),
      mode=lax.GatherScatterMode.PROMISE_IN_BOUNDS)


def _sc_body(xcat_hbm, xcont_hbm, table_hbm, w_hbm, bias_hbm,
             out_hbm, idx_v, gat_v, tr_v, xc_v, w_v, bias_v, cont_v,
             gsem0, gsem1, osem0, osem1, csem0, csem1):
  c = lax.axis_index("c")
  s = lax.axis_index("s")
  wid = s * NC + c
  base = wid * BW
  ctile0 = wid * NG
  gsems = (gsem0, gsem1)
  osems = (osem0, osem1)
  csems = (csem0, csem1)
  iota16 = lax.iota(jnp.int32, LANES)

  pltpu.sync_copy(w_hbm, w_v)
  pltpu.sync_copy(bias_hbm, bias_v)
  pltpu.sync_copy(xcont_hbm.at[:, pl.ds(base, BW)], xc_v)
  pltpu.sync_copy(xcat_hbm.at[:, pl.ds(base, BW)], idx_v)

  def add_body(f, carry):
    off = f * CARD
    for k in range(BW // LANES):
      sl = pl.ds(k * LANES, LANES)
      idx_v[f, sl] = idx_v[f, sl] + off
    return carry
  lax.fori_loop(0, N_CAT, add_body, 0)

  def gather_copy(st, p):
    f = st % N_CAT
    cg = st // N_CAT
    return pltpu.make_async_copy(
        table_hbm.at[idx_v.at[f, pl.ds(cg * GC, GC)]],
        gat_v.at[p], gsems[p])

  def out_copies(st, p):
    f = st % N_CAT
    cg = st // N_CAT
    return [
        pltpu.make_async_copy(
            tr_v.at[p, pl.ds(r * 8, 8), :],
            out_hbm.at[f, r, ctile0 + cg], osems[p])
        for r in range(SUB)
    ]

  def transpose_block(p):
    def d_body(d, carry):
      dsplat = jnp.full((LANES,), d, jnp.int32)
      psplat = jnp.full((LANES,), p, jnp.int32)
      for g in range(GC // LANES):
        vals = plsc.load_gather(
            gat_v, [psplat, iota16 + (g * LANES), dsplat])
        tr_v[p, d, pl.ds(g * LANES, LANES)] = vals
      return carry
    lax.fori_loop(0, DIM, d_body, 0)

  gather_copy(0, 0).start()

  def cat_body(sto, carry):
    for p in (0, 1):
      st = sto * 2 + p

      @pl.when(st + 1 < NST)
      def _():
        gather_copy(st + 1, 1 - p).start()

      @pl.when(st >= 2)
      def _():
        for cp in out_copies(st - 2, p):
          cp.wait()
      gather_copy(st, p).wait()
      transpose_block(p)
      for cp in out_copies(st, p):
        cp.start()
    return carry
  lax.fori_loop(0, NST // 2, cat_body, 0)

  for p in (0, 1):
    for cp in out_copies(NST - 2 + p, p):
      cp.wait()

  def cont_copies(st, p):
    j = st % N_CONT
    cg = st // N_CONT
    return [
        pltpu.make_async_copy(
            cont_v.at[p, pl.ds(r * 8, 8), :],
            out_hbm.at[N_CAT + j, r, ctile0 + cg], csems[p])
        for r in range(SUB)
    ]

  def cont_body(sto, carry):
    for p in (0, 1):
      st = sto * 2 + p
      j = st % N_CONT
      cg = st // N_CONT

      @pl.when(st >= 2)
      def _():
        for cp in cont_copies(st - 2, p):
          cp.wait()

      def d_body(d, carry2):
        dd = d // LANES
        wv = _splat(w_v[j, pl.ds(dd * LANES, LANES)], d % LANES)
        bv = _splat(bias_v[j, pl.ds(dd * LANES, LANES)], d % LANES)
        for k in range(GC // LANES):
          sl = pl.ds(k * LANES, LANES)
          cont_v[p, d, sl] = xc_v[j, pl.ds(cg * GC + k * LANES, LANES)] * wv + bv
        return carry2
      lax.fori_loop(0, DIM, d_body, 0)
      for cp in cont_copies(st, p):
        cp.start()
    return carry
  lax.fori_loop(0, (N_CONT * NG) // 2, cont_body, 0)

  for p in (0, 1):
    for cp in cont_copies(N_CONT * NG - 2 + p, p):
      cp.wait()


_sc_kernel = functools.partial(
    pl.kernel,
    mesh=plsc.VectorSubcoreMesh(core_axis_name="c", subcore_axis_name="s"),
    compiler_params=pltpu.CompilerParams(
        use_tc_tiling_on_sc=False, needs_layout_passes=False),
    out_type=jax.ShapeDtypeStruct((N_TOK, SUB, B // GC, 8, GC), jnp.float32),
    scratch_types=[
        pltpu.VMEM((N_CAT, BW), jnp.int32),
        pltpu.VMEM((2, GC, DIM), jnp.float32),
        pltpu.VMEM((2, DIM, GC), jnp.float32),
        pltpu.VMEM((N_CONT, BW), jnp.float32),
        pltpu.VMEM((N_CONT, DIM), jnp.float32),
        pltpu.VMEM((N_CONT, DIM), jnp.float32),
        pltpu.VMEM((2, DIM, GC), jnp.float32),
        pltpu.SemaphoreType.DMA,
        pltpu.SemaphoreType.DMA,
        pltpu.SemaphoreType.DMA,
        pltpu.SemaphoreType.DMA,
        pltpu.SemaphoreType.DMA,
        pltpu.SemaphoreType.DMA,
    ],
)(_sc_body)


@jax.jit
def kernel(x_cat, x_cont, cat_table, cont_W, cont_b):
  xcat_t = jnp.transpose(x_cat.astype(jnp.int32))
  xcont_t = jnp.transpose(x_cont)
  wide = lax.optimization_barrier(cat_table.reshape(N_CAT * CARD // 4, 128))
  table = wide.reshape(N_CAT * CARD, DIM)
  out5 = _sc_kernel(xcat_t, xcont_t, table, cont_W, cont_b)
  out_t = jnp.transpose(out5, (0, 1, 3, 2, 4)).reshape(N_TOK, DIM, B)
  return jnp.transpose(out_t, (2, 0, 1))
